# baseline (speedup 1.0000x reference)
import functools

import jax
import jax.numpy as jnp
from jax import lax
from jax.experimental import pallas as pl
from jax.experimental.pallas import tpu as pltpu
from jax.experimental.pallas import tpu_sc as plsc

B, K, L = 64, 32768, 16
NC, NS = 2, 16
NW = NC * NS
B_PER_W = B // NW
CK = 2048
NCHUNK = K // CK


def _scan_chunk(buf, k0, carry):
    def body(i, c):
        best, bidx = c
        v = buf[i]
        mask = v < best
        best = jnp.minimum(best, v)
        idx = jnp.full((L,), 0, jnp.int32) + (k0 + i)
        bidx = jnp.where(mask, idx, bidx)
        return best, bidx

    return lax.fori_loop(0, CK, body, carry, unroll=8)


@functools.partial(
    pl.kernel,
    out_type=jax.ShapeDtypeStruct((B, L), jnp.int32),
    mesh=plsc.VectorSubcoreMesh(core_axis_name="c", subcore_axis_name="s"),
    scratch_types=[
        pltpu.VMEM((CK, L), jnp.float32),
        pltpu.VMEM((B_PER_W, L), jnp.int32),
        pltpu.SemaphoreType.DMA,
    ],
    compiler_params=pltpu.CompilerParams(use_tc_tiling_on_sc=False),
)
def _argmin_sc(x_hbm, out_hbm, buf, outbuf, sem):
    wid = lax.axis_index("s") * NC + lax.axis_index("c")

    for b in range(B_PER_W):
        batch = wid * B_PER_W + b
        best = jnp.full((L,), jnp.inf, jnp.float32)
        bidx = jnp.full((L,), 0, jnp.int32)

        def chunk_body(c, carry):
            k0 = c * CK
            pltpu.async_copy(
                x_hbm.at[batch, pl.ds(k0, CK)], buf, sem
            ).wait()
            return _scan_chunk(buf, k0, carry)

        best, bidx = lax.fori_loop(0, NCHUNK, chunk_body, (best, bidx))
        outbuf[b] = bidx

    pltpu.sync_copy(outbuf, out_hbm.at[pl.ds(wid * B_PER_W, B_PER_W)])


def kernel(x):
    out32 = _argmin_sc(x)
    return out32.reshape(B, 1, L).astype(jnp.int64)


# speedup vs baseline: 1.0502x; 1.0502x over previous
import functools

import jax
import jax.numpy as jnp
from jax import lax
from jax.experimental import pallas as pl
from jax.experimental.pallas import tpu as pltpu
from jax.experimental.pallas import tpu_sc as plsc

B, K, L = 64, 32768, 16
NC, NS = 2, 16
NW = NC * NS
B_PER_W = B // NW
CK = 2048
NCHUNK = K // CK
NT = B_PER_W * NCHUNK


def _scan_chunk(buf, k0, carry):
    def body(i, c):
        best, bidx = c
        v = buf[pl.ds(i * L, L)]
        mask = v < best
        best = jnp.minimum(best, v)
        idx = jnp.full((L,), 0, jnp.int32) + (k0 + i)
        bidx = jnp.where(mask, idx, bidx)
        return best, bidx

    return lax.fori_loop(0, CK, body, carry, unroll=8)


@functools.partial(
    pl.kernel,
    out_type=jax.ShapeDtypeStruct((B, L), jnp.int32),
    mesh=plsc.VectorSubcoreMesh(core_axis_name="c", subcore_axis_name="s"),
    scratch_types=[
        pltpu.VMEM((CK * L,), jnp.float32),
        pltpu.VMEM((CK * L,), jnp.float32),
        pltpu.VMEM((B_PER_W, L), jnp.int32),
        pltpu.SemaphoreType.DMA,
        pltpu.SemaphoreType.DMA,
    ],
)
def _argmin_sc(x_hbm, out_hbm, buf0, buf1, outbuf, sem0, sem1):
    wid = lax.axis_index("s") * NC + lax.axis_index("c")
    base = wid * B_PER_W * K * L

    bufs = (buf0, buf1)
    sems = (sem0, sem1)

    def start(t):
        return pltpu.async_copy(
            x_hbm.at[pl.ds(base + t * CK * L, CK * L)], bufs[t % 2], sems[t % 2]
        )

    pending = start(0)
    best = bidx = None
    for t in range(NT):
        nxt = start(t + 1) if t + 1 < NT else None
        pending.wait()
        if t % NCHUNK == 0:
            best = jnp.full((L,), jnp.inf, jnp.float32)
            bidx = jnp.full((L,), 0, jnp.int32)
        best, bidx = _scan_chunk(bufs[t % 2], (t % NCHUNK) * CK, (best, bidx))
        if (t + 1) % NCHUNK == 0:
            outbuf[t // NCHUNK] = bidx
        pending = nxt

    pltpu.sync_copy(outbuf, out_hbm.at[pl.ds(wid * B_PER_W, B_PER_W)])


def kernel(x):
    out32 = _argmin_sc(x.reshape(-1))
    return out32.reshape(B, 1, L).astype(jnp.int64)


# speedup vs baseline: 10.0896x; 9.6078x over previous
import functools

import jax
import jax.numpy as jnp
from jax import lax
from jax.experimental import pallas as pl
from jax.experimental.pallas import tpu as pltpu
from jax.experimental.pallas import tpu_sc as plsc

B, K, CH = 64, 32768, 16
L = 16
NC, NS = 2, 16
NW = NC * NS
NBLK = B * 2
BLK_PER_W = NBLK // NW
KT = 256
TK = 32
NCHUNK = KT // TK
CHUNKS_PER_W = BLK_PER_W * NCHUNK
IMAX = 2**31 - 1


def _scan_chunk(buf, t0, carries):
    iota = lax.iota(jnp.int32, L)
    new = []
    for c in range(8):
        def tbody(t, carry, c=c):
            best, bidx = carry
            kbase = (t0 + t) * 128
            for j in range(8):
                v = buf[t, c, pl.ds(j * L, L)]
                idx = iota + (kbase + j * L)
                mask = v < best
                best = jnp.minimum(best, v)
                bidx = jnp.where(mask, idx, bidx)
            return best, bidx

        new.append(lax.fori_loop(0, TK, tbody, carries[c]))
    return new


@functools.partial(
    pl.kernel,
    out_type=jax.ShapeDtypeStruct((NBLK, L), jnp.int32),
    mesh=plsc.VectorSubcoreMesh(core_axis_name="c", subcore_axis_name="s"),
    scratch_types=[
        pltpu.VMEM((TK, 8, 128), jnp.float32),
        pltpu.VMEM((TK, 8, 128), jnp.float32),
        pltpu.VMEM((BLK_PER_W, L), jnp.int32),
        pltpu.SemaphoreType.DMA,
        pltpu.SemaphoreType.DMA,
    ],
    compiler_params=pltpu.CompilerParams(
        use_tc_tiling_on_sc=False, needs_layout_passes=False
    ),
)
def _argmin_sc(z_hbm, out_hbm, buf0, buf1, outbuf, sem0, sem1):
    wid = lax.axis_index("s") * NC + lax.axis_index("c")
    blk0 = wid * BLK_PER_W

    bufs = (buf0, buf1)
    sems = (sem0, sem1)

    def start(g, parity):
        gc = jnp.minimum(g, CHUNKS_PER_W - 1)
        blk = blk0 + gc // NCHUNK
        t0 = (gc % NCHUNK) * TK
        pltpu.async_copy(z_hbm.at[blk, pl.ds(t0, TK)], bufs[parity],
                         sems[parity])

    def wait(parity):
        pltpu.make_async_copy(
            z_hbm.at[0, pl.ds(0, TK)], bufs[parity], sems[parity]
        ).wait()

    start(0, 0)
    start(1, 1)

    for blk in range(BLK_PER_W):
        carries = [(jnp.full((L,), jnp.inf, jnp.float32),
                    jnp.full((L,), 0, jnp.int32)) for _ in range(8)]

        def pbody(p, flat, blk=blk):
            carries = [(flat[2 * c], flat[2 * c + 1]) for c in range(8)]
            for parity in range(2):
                g = blk * NCHUNK + 2 * p + parity
                wait(parity)
                carries = _scan_chunk(bufs[parity], (2 * p + parity) * TK,
                                      carries)
                start(g + 2, parity)
            return tuple(y for carry in carries for y in carry)

        flat = lax.fori_loop(0, NCHUNK // 2, pbody,
                             tuple(y for carry in carries for y in carry))

        lane = lax.iota(jnp.int32, L)
        acc = jnp.full((L,), 0, jnp.int32)
        for c in range(8):
            best, bidx = flat[2 * c], flat[2 * c + 1]
            m = lax.reduce_min(best, (0,))
            cand = jnp.where(best == m, bidx, IMAX)
            r = lax.reduce_min(cand, (0,))
            acc = jnp.where(lane == c, r, acc)
        outbuf[blk] = acc

    wait(0)
    wait(1)

    pltpu.sync_copy(outbuf, out_hbm.at[pl.ds(wid * BLK_PER_W, BLK_PER_W)])


def kernel(x):
    z = x.reshape(B, KT, 128, 2, 8).transpose(0, 3, 1, 4, 2)
    z = z.reshape(NBLK, KT, 8, 128)
    out32 = _argmin_sc(z)
    return out32[:, :8].reshape(B, 1, CH).astype(jnp.int64)
